# Initial kernel scaffold; baseline (speedup 1.0000x reference)
import jax
import jax.numpy as jnp
from jax.experimental import pallas as pl


def kernel(confidence, predicted_locations, labels, gt_locations):
    raise NotImplementedError("write your pallas kernel here")


# speedup vs baseline: 6.4454x; 6.4454x over previous
import math

import jax
import jax.numpy as jnp
from jax.experimental import pallas as pl

NEG_POS_RATIO_ = 3
NEG_LAMBDA_ = 0.1 / math.log(3.5)
INV_NEG_LAMBDA_ = 1.0 / NEG_LAMBDA_
TAU_ = 4.0
L_ = 6.0
MARGIN_ = 0.5


def _stage_a(conf_ref, lab_ref, pred_ref, gt_ref,
             lossn_ref, sn_ref, wn_ref, acc_ref):
    conf = conf_ref[0]
    lab = lab_ref[0]
    C = conf.shape[0]

    m = jnp.max(conf, axis=0, keepdims=True)
    ex = jnp.exp(conf - m)
    lse = m + jnp.log(jnp.sum(ex, axis=0, keepdims=True))

    sig = 1.0 / (1.0 + jnp.exp(-conf))
    e = jnp.exp((sig - 1.0) * INV_NEG_LAMBDA_)
    s_all = jnp.sum(e, axis=0, keepdims=True)
    w_all = jnp.sum(e * sig, axis=0, keepdims=True)

    cls = jax.lax.broadcasted_iota(jnp.int32, (C, conf.shape[1]), 0)
    is_lab = cls == lab
    is_lab1 = cls == (lab - 1)
    conf_l = jnp.sum(jnp.where(is_lab, conf, 0.0), axis=0, keepdims=True)
    e_l1 = jnp.sum(jnp.where(is_lab1, e, 0.0), axis=0, keepdims=True)
    sig_l1 = jnp.sum(jnp.where(is_lab1, sig, 0.0), axis=0, keepdims=True)

    pos = lab > 0
    posf = pos.astype(jnp.float32)
    loss = lse - conf[0:1, :]

    lossn_ref[0] = jnp.where(pos, -jnp.inf, loss)
    sn_ref[0] = jnp.where(pos, 0.0, s_all)
    wn_ref[0] = jnp.where(pos, 0.0, w_all)

    ep = jnp.exp(-sig_l1)
    class_p = jnp.sum(posf * (lse - conf_l))
    s_neg_p = jnp.sum(posf * (s_all - e_l1))
    w_neg_p = jnp.sum(posf * (w_all - e_l1 * sig_l1))
    s_pos = jnp.sum(posf * ep)
    w_pos = jnp.sum(posf * sig_l1 * ep)
    n_pos = jnp.sum(posf)

    d = jnp.abs(pred_ref[0] - gt_ref[0])
    h = jnp.where(d < 1.0, 0.5 * d * d, d - 0.5)
    sl1 = jnp.sum(posf * jnp.sum(h, axis=0, keepdims=True))

    lane = jax.lax.broadcasted_iota(jnp.int32, (1, 128), 1)
    packed = (class_p * (lane == 0) + s_neg_p * (lane == 1)
              + w_neg_p * (lane == 2) + s_pos * (lane == 3)
              + w_pos * (lane == 4) + sl1 * (lane == 5)
              + n_pos * (lane == 6))
    acc_ref[0] = packed.astype(jnp.float32)


def _stage_b(lossn_ref, sn_ref, wn_ref, acc_ref, out_ref):
    loss = lossn_ref[...]
    sn = sn_ref[...]
    wn = wn_ref[...]
    acc = acc_ref[...]
    B, P = loss.shape

    bits = jax.lax.bitcast_convert_type(loss, jnp.int32)
    keys = jnp.where(bits >= 0, bits, bits ^ jnp.int32(0x7FFFFFFF))
    hi = keys >= 0
    c1 = jnp.sum(hi.astype(jnp.int32), axis=1, keepdims=True)

    k = (acc[:, 6:7] * float(NEG_POS_RATIO_)).astype(jnp.int32)
    group1 = k <= c1
    kk = jnp.where(group1, k, k - c1)
    ingrp = hi == group1
    lo = keys & jnp.int32(0x7FFFFFFF)

    T = jnp.zeros((B, 1), jnp.int32)
    for bit in range(30, -1, -1):
        probe = T | jnp.int32(1 << bit)
        cnt = jnp.sum(jnp.where(ingrp & (lo >= probe), 1, 0),
                      axis=1, keepdims=True)
        T = jnp.where(cnt >= kk, probe, T)
    kstar = jnp.where(group1, T, T + jnp.int32(-2147483648))

    gt_m = keys > kstar
    cgt = jnp.sum(gt_m.astype(jnp.int32), axis=1, keepdims=True)
    take = k - cgt
    eq = keys == kstar
    eqi = eq.astype(jnp.int32)
    pref = eqi
    s = 1
    while s < P:
        pref = pref + jnp.concatenate(
            [jnp.zeros((B, s), jnp.int32), pref[:, :P - s]], axis=1)
        s *= 2
    excl = pref - eqi
    sel = gt_m | (eq & (excl < take))

    s_neg_n = jnp.sum(jnp.where(sel, sn, 0.0))
    w_neg_n = jnp.sum(jnp.where(sel, wn, 0.0))
    class_n = jnp.sum(jnp.where(sel & (loss != -jnp.inf), loss, 0.0))

    tot = jnp.sum(acc, axis=0, keepdims=True)
    class_p = tot[0, 0]
    s_neg_p = tot[0, 1]
    w_neg_p = tot[0, 2]
    s_pos = tot[0, 3]
    w_pos = tot[0, 4]
    sl1 = tot[0, 5]
    n_pos = tot[0, 6]

    neg_dist = (w_neg_p + w_neg_n) / (s_neg_p + s_neg_n)
    pos_dist = w_pos / s_pos
    diff = jnp.where(n_pos > 0.0, neg_dist - pos_dist, neg_dist - 1.0)
    drl = TAU_ * jnp.log(1.0 + jnp.exp(L_ * (diff + MARGIN_))) / L_

    lane = jax.lax.broadcasted_iota(jnp.int32, (1, 128), 1)
    out = ((sl1 / n_pos) * (lane == 0) + drl * (lane == 1)
           + ((class_p + class_n) / n_pos) * (lane == 2))
    out_ref[...] = out.astype(jnp.float32)


def kernel(confidence, predicted_locations, labels, gt_locations):
    B, P, C = confidence.shape
    conf_t = jnp.transpose(confidence, (0, 2, 1))
    pred_t = jnp.transpose(predicted_locations, (0, 2, 1))
    gt_t = jnp.transpose(gt_locations, (0, 2, 1))
    lab3 = labels.astype(jnp.int32).reshape(B, 1, P)

    lossn, sn, wn, acc = pl.pallas_call(
        _stage_a,
        grid=(B,),
        in_specs=[
            pl.BlockSpec((1, C, P), lambda b: (b, 0, 0)),
            pl.BlockSpec((1, 1, P), lambda b: (b, 0, 0)),
            pl.BlockSpec((1, 4, P), lambda b: (b, 0, 0)),
            pl.BlockSpec((1, 4, P), lambda b: (b, 0, 0)),
        ],
        out_specs=[
            pl.BlockSpec((1, 1, P), lambda b: (b, 0, 0)),
            pl.BlockSpec((1, 1, P), lambda b: (b, 0, 0)),
            pl.BlockSpec((1, 1, P), lambda b: (b, 0, 0)),
            pl.BlockSpec((1, 1, 128), lambda b: (b, 0, 0)),
        ],
        out_shape=[
            jax.ShapeDtypeStruct((B, 1, P), jnp.float32),
            jax.ShapeDtypeStruct((B, 1, P), jnp.float32),
            jax.ShapeDtypeStruct((B, 1, P), jnp.float32),
            jax.ShapeDtypeStruct((B, 1, 128), jnp.float32),
        ],
    )(conf_t, lab3, pred_t, gt_t)

    out = pl.pallas_call(
        _stage_b,
        out_shape=jax.ShapeDtypeStruct((1, 128), jnp.float32),
    )(lossn.reshape(B, P), sn.reshape(B, P), wn.reshape(B, P),
      acc.reshape(B, 128))

    return (out[0, 0], out[0, 1], out[0, 2])
